# baseline (speedup 1.0000x reference)
import functools

import jax
import jax.numpy as jnp
from jax import lax
from jax.experimental import pallas as pl
from jax.experimental.pallas import tpu as pltpu
from jax.experimental.pallas import tpu_sc as plsc


def _dequant_body(w_ref, s_ref, o_ref):
    o_ref[...] = w_ref[...].astype(jnp.float32) * s_ref[...]


def _dequantize(w8, s, blk):
    rows, d = w8.shape
    return pl.pallas_call(
        _dequant_body,
        grid=(rows // blk,),
        in_specs=[
            pl.BlockSpec((blk, d), lambda i: (i, 0)),
            pl.BlockSpec((blk, 1), lambda i: (i, 0)),
        ],
        out_specs=pl.BlockSpec((blk, d), lambda i: (i, 0)),
        out_shape=jax.ShapeDtypeStruct((rows, d), jnp.float32),
    )(w8, s)


def _sc_gather(table, idx_flat, d):
    info = plsc.get_sparse_core_info()
    nc, ns = info.num_cores, info.num_subcores
    nw = nc * ns
    n = idx_flat.shape[0]
    per_w = n // nw
    ch = 1600
    n_ch = per_w // ch
    mesh = plsc.VectorSubcoreMesh(core_axis_name="c", subcore_axis_name="s")

    @functools.partial(
        pl.kernel,
        mesh=mesh,
        out_type=jax.ShapeDtypeStruct((n, d), jnp.float32),
        scratch_types=[
            pltpu.VMEM((ch,), jnp.int32),
            pltpu.VMEM((ch, d), jnp.float32),
            pltpu.SemaphoreType.DMA,
        ],
        compiler_params=pltpu.CompilerParams(use_tc_tiling_on_sc=False),
    )
    def k(table_hbm, idx_hbm, out_hbm, idx_v, rows_v, sem):
        wid = lax.axis_index("s") * nc + lax.axis_index("c")
        base = wid * per_w

        def body(i, carry):
            off = base + i * ch
            pltpu.sync_copy(idx_hbm.at[pl.ds(off, ch)], idx_v)
            pltpu.async_copy(table_hbm.at[idx_v], rows_v, sem).wait()
            pltpu.sync_copy(rows_v, out_hbm.at[pl.ds(off, ch)])
            return carry

        lax.fori_loop(0, n_ch, body, 0)

    return k(table, idx_flat)


def kernel(input, weight, weight_scale):
    v, d = weight.shape
    table = _dequantize(weight, weight_scale[:, None], blk=8000)
    idx = input.reshape(-1)
    out = _sc_gather(table, idx, d)
    return out.reshape(*input.shape, d)


# speedup vs baseline: 1.8749x; 1.8749x over previous
import functools

import jax
import jax.numpy as jnp
from jax import lax
from jax.experimental import pallas as pl
from jax.experimental.pallas import tpu as pltpu
from jax.experimental.pallas import tpu_sc as plsc


def _dequant_body(w_ref, s_ref, o_ref):
    rows = lax.broadcasted_iota(jnp.int32, (8, 128), 0)
    cols = lax.broadcasted_iota(jnp.int32, (8, 128), 1)
    expand = (cols // 16 == rows).astype(jnp.float32)
    s128 = jnp.dot(s_ref[...], expand, preferred_element_type=jnp.float32)
    o_ref[...] = w_ref[...].astype(jnp.float32) * s128


def _dequantize(w128, s8, blk):
    rows, lanes = w128.shape
    return pl.pallas_call(
        _dequant_body,
        grid=(rows // blk,),
        in_specs=[
            pl.BlockSpec((blk, lanes), lambda i: (i, 0)),
            pl.BlockSpec((blk, 8), lambda i: (i, 0)),
        ],
        out_specs=pl.BlockSpec((blk, lanes), lambda i: (i, 0)),
        out_shape=jax.ShapeDtypeStruct((rows, lanes), jnp.float32),
    )(w128, s8)


def _sc_gather(table, idx_flat, d):
    info = plsc.get_sparse_core_info()
    nc, ns = info.num_cores, info.num_subcores
    nw = nc * ns
    n = idx_flat.shape[0]
    per_w = n // nw
    ch = 1600
    n_ch = per_w // ch
    nbuf = 3
    mesh = plsc.VectorSubcoreMesh(core_axis_name="c", subcore_axis_name="s")

    @functools.partial(
        pl.kernel,
        mesh=mesh,
        out_type=jax.ShapeDtypeStruct((n, d), jnp.float32),
        scratch_types=[pltpu.VMEM((per_w,), jnp.int32)]
        + [pltpu.VMEM((ch, d), jnp.float32) for _ in range(nbuf)]
        + [pltpu.SemaphoreType.DMA for _ in range(2 * nbuf)],
        compiler_params=pltpu.CompilerParams(use_tc_tiling_on_sc=False),
    )
    def k(table_hbm, idx_hbm, out_hbm, idx_v, *bufs_and_sems):
        rows = list(bufs_and_sems[:nbuf])
        gsem = list(bufs_and_sems[nbuf : 2 * nbuf])
        wsem = list(bufs_and_sems[2 * nbuf :])
        wid = lax.axis_index("s") * nc + lax.axis_index("c")
        base = wid * per_w

        pltpu.sync_copy(idx_hbm.at[pl.ds(base, per_w)], idx_v)

        gh = [None] * nbuf
        wh = [None] * nbuf
        for g in range(nbuf):
            gh[g] = pltpu.async_copy(
                table_hbm.at[idx_v.at[pl.ds(g * ch, ch)]], rows[g], gsem[g]
            )
        for g in range(n_ch):
            b = g % nbuf
            gh[b].wait()
            wh[b] = pltpu.async_copy(
                rows[b], out_hbm.at[pl.ds(base + g * ch, ch)], wsem[b]
            )
            ng = g + nbuf
            if ng < n_ch:
                wh[b].wait()
                gh[b] = pltpu.async_copy(
                    table_hbm.at[idx_v.at[pl.ds(ng * ch, ch)]], rows[b], gsem[b]
                )
        for g in range(n_ch - nbuf, n_ch):
            wh[g % nbuf].wait()

    return k(table, idx_flat)


def kernel(input, weight, weight_scale):
    v, d = weight.shape
    w128 = weight.reshape(v // 8, 8 * d)
    s8 = weight_scale.reshape(v // 8, 8)
    table128 = _dequantize(w128, s8, blk=5000)
    table = table128.reshape(v, d)
    idx = input.reshape(-1)
    out = _sc_gather(table, idx, d)
    return out.reshape(*input.shape, d)
